# baseline (speedup 1.0000x reference)
import functools

import jax
import jax.numpy as jnp
from jax import lax
from jax.experimental import pallas as pl
from jax.experimental.pallas import tpu as pltpu
from jax.experimental.pallas import tpu_sc as plsc

BATCH = 16384
HIST = 50
DIM = 64

NC = 2
NS = 16
NW = NC * NS
BPW = BATCH // NW
CB = 128
NCH = BPW // CB
L = 16


def _gather_kernel(idx_hbm, table_hbm, out_hbm, idx_v, gidx, par, stag, trans,
                   gsem, ssem):
    wid = lax.axis_index("s") * NC + lax.axis_index("c")
    b0 = wid * BPW
    lane = lax.iota(jnp.int32, L)

    def h_body(h, carry):
        pltpu.sync_copy(idx_hbm.at[pl.ds(h, 1), pl.ds(b0, BPW)], idx_v)
        for c16 in range(BPW // L):
            iv = idx_v[0, pl.ds(c16 * L, L)]
            gidx[c16 // 8, pl.ds((c16 % 8) * L, L)] = lax.shift_right_logical(iv, 1)
            par[c16 // 8, pl.ds((c16 % 8) * L, L)] = (iv & 1) * DIM

        for c in range(NCH):
            pltpu.async_copy(table_hbm.at[gidx.at[c]], stag, gsem).wait()
            for b16 in range(CB // L):
                rows = lane + b16 * L
                cols0 = par[c, pl.ds(b16 * L, L)]
                for d in range(DIM):
                    trans[d, pl.ds(b16 * L, L)] = plsc.load_gather(
                        stag, [rows, cols0 + d]
                    )
            pltpu.async_copy(
                trans, out_hbm.at[h, :, pl.ds(b0 + c * CB, CB)], ssem
            ).wait()
        return carry

    lax.fori_loop(0, HIST, h_body, 0)


def kernel(inp, table):
    idx_t = jnp.transpose(inp.astype(jnp.int32), (1, 0))
    tab128 = jnp.reshape(table, (table.shape[0] // 2, 2 * DIM))
    mesh = plsc.VectorSubcoreMesh(core_axis_name="c", subcore_axis_name="s")
    run = functools.partial(
        pl.kernel,
        mesh=mesh,
        out_type=jax.ShapeDtypeStruct((HIST, DIM, BATCH), jnp.float32),
        scratch_types=[
            pltpu.VMEM((1, BPW), jnp.int32),
            pltpu.VMEM((NCH, CB), jnp.int32),
            pltpu.VMEM((NCH, CB), jnp.int32),
            pltpu.VMEM((CB, 2 * DIM), jnp.float32),
            pltpu.VMEM((DIM, CB), jnp.float32),
            pltpu.SemaphoreType.DMA,
            pltpu.SemaphoreType.DMA,
        ],
        compiler_params=pltpu.CompilerParams(
            use_tc_tiling_on_sc=True, needs_layout_passes=False
        ),
    )(_gather_kernel)
    out_t = run(idx_t, tab128)
    return jnp.transpose(out_t, (2, 0, 1))


# speedup vs baseline: 1.1892x; 1.1892x over previous
import functools

import jax
import jax.numpy as jnp
from jax import lax
from jax.experimental import pallas as pl
from jax.experimental.pallas import tpu as pltpu
from jax.experimental.pallas import tpu_sc as plsc

BATCH = 16384
HIST = 50
DIM = 64

NC = 2
NS = 16
NW = NC * NS
BPW = BATCH // NW
CB = 128
NCH = BPW // CB
NG = HIST * NCH
L = 16
VPR = BPW // L


def _gather_kernel(idx_hbm, table_hbm, out_hbm, par, gidx, stag, trans,
                   gsem, ssem):
    wid = lax.axis_index("s") * NC + lax.axis_index("c")
    b0 = wid * BPW
    lane = lax.iota(jnp.int32, L)

    pltpu.sync_copy(idx_hbm.at[:, pl.ds(b0, BPW)], par)
    for r in range(HIST):
        for v in range(VPR):
            iv = par[r, pl.ds(v * L, L)]
            gidx[r * NCH + v // 8, pl.ds((v % 8) * L, L)] = (
                lax.shift_right_logical(iv, 1))
            par[r, pl.ds(v * L, L)] = (iv & 1) * DIM

    def fire_gather(g, buf):
        pltpu.async_copy(table_hbm.at[gidx.at[g]], stag.at[buf], gsem)

    def wait_gather(buf):
        pltpu.make_async_copy(table_hbm.at[gidx.at[0]], stag.at[buf], gsem).wait()

    def fire_store(g, buf):
        h = g // NCH
        c = lax.rem(g, NCH)
        pltpu.async_copy(
            trans.at[buf], out_hbm.at[h, :, pl.ds(b0 + c * CB, CB)], ssem)

    def wait_store():
        pltpu.make_async_copy(
            trans.at[0], out_hbm.at[0, :, pl.ds(b0, CB)], ssem).wait()

    def transpose(g, buf):
        h = g // NCH
        c = lax.rem(g, NCH)
        hv = jnp.broadcast_to(h, (L,)).astype(jnp.int32)
        for b16 in range(CB // L):
            rows = lane + b16 * L
            cols0 = plsc.load_gather(par, [hv, c * CB + b16 * L + lane])
            for d in range(DIM):
                trans[buf, d, pl.ds(b16 * L, L)] = plsc.load_gather(
                    stag.at[buf], [rows, cols0 + d])

    fire_gather(0, 0)

    def body(p, carry):
        for sub in range(2):
            g = p * 2 + sub
            wait_gather(sub)

            @pl.when(g < NG - 1)
            def _():
                fire_gather(g + 1, 1 - sub)

            @pl.when(g >= 2)
            def _():
                wait_store()

            transpose(g, sub)
            fire_store(g, sub)
        return carry

    lax.fori_loop(0, NG // 2, body, 0)
    wait_store()
    wait_store()


def kernel(inp, table):
    idx_t = jnp.transpose(inp.astype(jnp.int32), (1, 0))
    tab128 = jnp.reshape(table, (table.shape[0] // 2, 2 * DIM))
    mesh = plsc.VectorSubcoreMesh(core_axis_name="c", subcore_axis_name="s")
    run = functools.partial(
        pl.kernel,
        mesh=mesh,
        out_type=jax.ShapeDtypeStruct((HIST, DIM, BATCH), jnp.float32),
        scratch_types=[
            pltpu.VMEM((HIST, BPW), jnp.int32),
            pltpu.VMEM((NG, CB), jnp.int32),
            pltpu.VMEM((2, CB, 2 * DIM), jnp.float32),
            pltpu.VMEM((2, DIM, CB), jnp.float32),
            pltpu.SemaphoreType.DMA,
            pltpu.SemaphoreType.DMA,
        ],
        compiler_params=pltpu.CompilerParams(
            use_tc_tiling_on_sc=True, needs_layout_passes=False
        ),
    )(_gather_kernel)
    out_t = run(idx_t, tab128)
    return jnp.transpose(out_t, (2, 0, 1))


# speedup vs baseline: 1.1894x; 1.0002x over previous
import functools

import jax
import jax.numpy as jnp
from jax import lax
from jax.experimental import pallas as pl
from jax.experimental.pallas import tpu as pltpu
from jax.experimental.pallas import tpu_sc as plsc

BATCH = 16384
HIST = 50
DIM = 64

NC = 2
NS = 16
NW = NC * NS
BPW = BATCH // NW
CB = 128
NCH = BPW // CB
NG = HIST * NCH
L = 16
VPR = BPW // L


def _gather_kernel(idx_hbm, table_hbm, out_hbm, par, gidx, stag, trans,
                   gsem, ssem):
    wid = lax.axis_index("s") * NC + lax.axis_index("c")
    b0 = wid * BPW
    lane = lax.iota(jnp.int32, L)

    pltpu.sync_copy(idx_hbm.at[:, pl.ds(b0, BPW)], par)
    for r in range(HIST):
        for v in range(VPR):
            iv = par[r, pl.ds(v * L, L)]
            gidx[r * NCH + v // 8, pl.ds((v % 8) * L, L)] = (
                lax.shift_right_logical(iv, 1))
            par[r, pl.ds(v * L, L)] = (iv & 1) * DIM

    def fire_gather(g, buf):
        pltpu.async_copy(table_hbm.at[gidx.at[g]], stag.at[buf], gsem)

    def wait_gather(buf):
        pltpu.make_async_copy(table_hbm.at[gidx.at[0]], stag.at[buf], gsem).wait()

    def fire_store(g, buf):
        h = g // NCH
        c = lax.rem(g, NCH)
        pltpu.async_copy(
            trans.at[buf], out_hbm.at[h, :, pl.ds(b0 + c * CB, CB)], ssem)

    def wait_store():
        pltpu.make_async_copy(
            trans.at[0], out_hbm.at[0, :, pl.ds(b0, CB)], ssem).wait()

    def transpose(g, buf):
        h = g // NCH
        c = lax.rem(g, NCH)
        hv = jnp.broadcast_to(h, (L,)).astype(jnp.int32)
        for b16 in range(CB // L):
            rows = lane + b16 * L
            cols0 = plsc.load_gather(par, [hv, c * CB + b16 * L + lane])
            for d in range(DIM):
                trans[buf, d, pl.ds(b16 * L, L)] = plsc.load_gather(
                    stag.at[buf], [rows, cols0 + d])

    fire_gather(0, 0)

    def body(p, carry):
        for sub in range(2):
            g = p * 2 + sub
            wait_gather(sub)

            @pl.when(g < NG - 1)
            def _():
                fire_gather(g + 1, 1 - sub)

            @pl.when(g >= 2)
            def _():
                wait_store()

            transpose(g, sub)
            fire_store(g, sub)
        return carry

    lax.fori_loop(0, NG // 2, body, 0)
    wait_store()
    wait_store()


def kernel(inp, table):
    idx_t = jnp.transpose(inp.astype(jnp.int32), (1, 0))
    tab128 = jnp.reshape(table, (table.shape[0] // 2, 2 * DIM))
    mesh = plsc.VectorSubcoreMesh(core_axis_name="c", subcore_axis_name="s")
    run = functools.partial(
        pl.kernel,
        mesh=mesh,
        out_type=jax.ShapeDtypeStruct((HIST, DIM, BATCH), jnp.float32),
        scratch_types=[
            pltpu.VMEM((HIST, BPW), jnp.int32),
            pltpu.VMEM((NG, CB), jnp.int32),
            pltpu.VMEM((2, CB, 2 * DIM), jnp.float32),
            pltpu.VMEM((2, DIM, CB), jnp.float32),
            pltpu.SemaphoreType.DMA,
            pltpu.SemaphoreType.DMA,
        ],
        compiler_params=pltpu.CompilerParams(
            use_tc_tiling_on_sc=True,
            needs_layout_passes=False,
            disable_bounds_checks=True,
        ),
    )(_gather_kernel)
    out_t = run(idx_t, tab128)
    return jnp.transpose(out_t, (2, 0, 1))


# speedup vs baseline: 1.8348x; 1.5426x over previous
import functools

import jax
import jax.numpy as jnp
from jax import lax
from jax.experimental import pallas as pl
from jax.experimental.pallas import tpu as pltpu
from jax.experimental.pallas import tpu_sc as plsc

BATCH = 16384
HIST = 50
DIM = 64

NC = 2
NS = 16
NW = NC * NS
BPW = BATCH // NW
CB = 128
NCH = BPW // CB
NG = HIST * NCH
L = 16
VPR = BPW // L


def _gather_kernel(idx_hbm, table_hbm, out_hbm, par, gidx, stag, trans,
                   gsem, ssem):
    wid = lax.axis_index("s") * NC + lax.axis_index("c")
    b0 = wid * BPW
    lane = lax.iota(jnp.int32, L)

    pltpu.sync_copy(idx_hbm.at[:, pl.ds(b0, BPW)], par)
    for r in range(HIST):
        for v in range(VPR):
            iv = par[r, pl.ds(v * L, L)]
            gidx[r * NCH + v // 8, pl.ds((v % 8) * L, L)] = (
                lax.shift_right_logical(iv, 1))
            par[r, pl.ds(v * L, L)] = (iv & 1) * DIM

    def fire_gather(g, buf):
        pltpu.async_copy(table_hbm.at[gidx.at[g]], stag.at[buf], gsem)

    def wait_gather(buf):
        pltpu.make_async_copy(table_hbm.at[gidx.at[0]], stag.at[buf], gsem).wait()

    def fire_store(g, buf):
        h = g // NCH
        c = lax.rem(g, NCH)
        pltpu.async_copy(
            trans.at[buf], out_hbm.at[h, :, pl.ds(b0 + c * CB, CB)], ssem)

    def wait_store():
        pltpu.make_async_copy(
            trans.at[0], out_hbm.at[0, :, pl.ds(b0, CB)], ssem).wait()

    zero = jnp.broadcast_to(jnp.int32(0), (L,))

    def transpose(g, buf):
        h = g // NCH
        c = lax.rem(g, NCH)
        hv = jnp.broadcast_to(h, (L,)).astype(jnp.int32)
        base = [
            (lane + b16 * L) * CB
            | plsc.load_gather(par, [hv, c * CB + b16 * L + lane])
            for b16 in range(CB // L)
        ]
        @plsc.parallel_loop(0, DIM, step=1, unroll=8)
        def _(d):
            for b16 in range(CB // L):
                trans[buf, d, pl.ds(b16 * L, L)] = plsc.load_gather(
                    stag.at[buf], [zero, base[b16] + d])

    fire_gather(0, 0)

    def body(p, carry):
        for sub in range(2):
            g = p * 2 + sub
            wait_gather(sub)

            @pl.when(g < NG - 1)
            def _():
                fire_gather(g + 1, 1 - sub)

            @pl.when(g >= 2)
            def _():
                wait_store()

            transpose(g, sub)
            fire_store(g, sub)
        return carry

    lax.fori_loop(0, NG // 2, body, 0)
    wait_store()
    wait_store()


def kernel(inp, table):
    idx_t = jnp.transpose(inp.astype(jnp.int32), (1, 0))
    tab128 = jnp.reshape(table, (table.shape[0] // 2, 2 * DIM))
    mesh = plsc.VectorSubcoreMesh(core_axis_name="c", subcore_axis_name="s")
    run = functools.partial(
        pl.kernel,
        mesh=mesh,
        out_type=jax.ShapeDtypeStruct((HIST, DIM, BATCH), jnp.float32),
        scratch_types=[
            pltpu.VMEM((HIST, BPW), jnp.int32),
            pltpu.VMEM((NG, CB), jnp.int32),
            pltpu.VMEM((2, CB, 2 * DIM), jnp.float32),
            pltpu.VMEM((2, DIM, CB), jnp.float32),
            pltpu.SemaphoreType.DMA,
            pltpu.SemaphoreType.DMA,
        ],
        compiler_params=pltpu.CompilerParams(
            use_tc_tiling_on_sc=True,
            needs_layout_passes=False,
            disable_bounds_checks=True,
        ),
    )(_gather_kernel)
    out_t = run(idx_t, tab128)
    return jnp.transpose(out_t, (2, 0, 1))


# speedup vs baseline: 1.9794x; 1.0788x over previous
import functools

import jax
import jax.numpy as jnp
from jax import lax
from jax.experimental import pallas as pl
from jax.experimental.pallas import tpu as pltpu
from jax.experimental.pallas import tpu_sc as plsc

BATCH = 16384
HIST = 50
DIM = 64

NC = 2
NS = 16
NW = NC * NS
BPW = BATCH // NW
CB = 128
NCH = BPW // CB
NG = HIST * NCH
L = 16
VPR = BPW // L


def _gather_kernel(idx_hbm, table_hbm, out_hbm, par, gidx, stag, trans,
                   gsem, ssem):
    wid = lax.axis_index("s") * NC + lax.axis_index("c")
    b0 = wid * BPW
    lane = lax.iota(jnp.int32, L)

    pltpu.sync_copy(idx_hbm.at[:, pl.ds(b0, BPW)], par)
    for r in range(HIST):
        for v in range(VPR):
            iv = par[r, pl.ds(v * L, L)]
            gidx[r * NCH + v // 8, pl.ds((v % 8) * L, L)] = (
                lax.shift_right_logical(iv, 1))
            par[r, pl.ds(v * L, L)] = (iv & 1) * DIM

    def fire_gather(g, buf):
        pltpu.async_copy(table_hbm.at[gidx.at[g]], stag.at[buf], gsem)

    def wait_gather(buf):
        pltpu.make_async_copy(table_hbm.at[gidx.at[0]], stag.at[buf], gsem).wait()

    def fire_store(g, buf):
        h = g // NCH
        c = lax.rem(g, NCH)
        pltpu.async_copy(
            trans.at[buf], out_hbm.at[h, :, pl.ds(b0 + c * CB, CB)], ssem)

    def wait_store():
        pltpu.make_async_copy(
            trans.at[0], out_hbm.at[0, :, pl.ds(b0, CB)], ssem).wait()

    zero = jnp.broadcast_to(jnp.int32(0), (L,))

    def transpose(g, buf):
        h = g // NCH
        c = lax.rem(g, NCH)
        hv = jnp.broadcast_to(h, (L,)).astype(jnp.int32)
        base = [
            (lane + b16 * L) * CB
            | plsc.load_gather(par, [hv, c * CB + b16 * L + lane])
            for b16 in range(CB // L)
        ]
        @plsc.parallel_loop(0, DIM, step=1, unroll=16)
        def _(d):
            for b16 in range(CB // L):
                trans[buf, d, pl.ds(b16 * L, L)] = plsc.load_gather(
                    stag.at[buf], [zero, base[b16] + d])

    fire_gather(0, 0)

    def body(p, carry):
        for sub in range(2):
            g = p * 2 + sub
            wait_gather(sub)

            @pl.when(g < NG - 1)
            def _():
                fire_gather(g + 1, 1 - sub)

            @pl.when(g >= 2)
            def _():
                wait_store()

            transpose(g, sub)
            fire_store(g, sub)
        return carry

    lax.fori_loop(0, NG // 2, body, 0)
    wait_store()
    wait_store()


def kernel(inp, table):
    idx_t = jnp.transpose(inp.astype(jnp.int32), (1, 0))
    tab128 = jnp.reshape(table, (table.shape[0] // 2, 2 * DIM))
    mesh = plsc.VectorSubcoreMesh(core_axis_name="c", subcore_axis_name="s")
    run = functools.partial(
        pl.kernel,
        mesh=mesh,
        out_type=jax.ShapeDtypeStruct((HIST, DIM, BATCH), jnp.float32),
        scratch_types=[
            pltpu.VMEM((HIST, BPW), jnp.int32),
            pltpu.VMEM((NG, CB), jnp.int32),
            pltpu.VMEM((2, CB, 2 * DIM), jnp.float32),
            pltpu.VMEM((2, DIM, CB), jnp.float32),
            pltpu.SemaphoreType.DMA,
            pltpu.SemaphoreType.DMA,
        ],
        compiler_params=pltpu.CompilerParams(
            use_tc_tiling_on_sc=True,
            needs_layout_passes=False,
            disable_bounds_checks=True,
        ),
    )(_gather_kernel)
    out_t = run(idx_t, tab128)
    return jnp.transpose(out_t, (2, 0, 1))
